# baseline (speedup 1.0000x reference)
import functools

import jax
import jax.numpy as jnp
from jax import lax
from jax.experimental import pallas as pl
from jax.experimental.pallas import tpu as pltpu
from jax.experimental.pallas import tpu_sc as plsc

EMBED_DIM = 64
SCALE = float(EMBED_DIM) ** 0.5

NUM_CORES = 2
NUM_SUBCORES = 16
NUM_WORKERS = NUM_CORES * NUM_SUBCORES
CHUNK = 128


def _make_lookup(n_total: int, vocab: int):
    per_worker = n_total // NUM_WORKERS
    n_chunks = per_worker // CHUNK

    mesh = plsc.VectorSubcoreMesh(
        core_axis_name="c", subcore_axis_name="s",
        num_cores=NUM_CORES, num_subcores=NUM_SUBCORES)

    @functools.partial(
        pl.kernel,
        out_type=jax.ShapeDtypeStruct((n_total, EMBED_DIM), jnp.float32),
        mesh=mesh,
        scratch_types=[
            pltpu.VMEM((n_chunks, CHUNK), jnp.int32),
            pltpu.VMEM((CHUNK, EMBED_DIM), jnp.float32),
            pltpu.SemaphoreType.DMA,
        ],
        compiler_params=pltpu.CompilerParams(use_tc_tiling_on_sc=False),
    )
    def lookup(x_hbm, table_hbm, out_hbm, idx_v, rows_v, gsem):
        wid = lax.axis_index("s") * NUM_CORES + lax.axis_index("c")
        base = wid * per_worker
        pltpu.sync_copy(x_hbm.at[wid], idx_v)

        def chunk_body(j, _):
            pltpu.async_copy(table_hbm.at[idx_v.at[j]], rows_v, gsem).wait()

            def scale_row(r, _):
                for k in range(EMBED_DIM // 16):
                    sl = pl.ds(k * 16, 16)
                    rows_v[r, sl] = rows_v[r, sl] * SCALE
                return ()

            lax.fori_loop(0, CHUNK, scale_row, ())
            pltpu.sync_copy(rows_v, out_hbm.at[pl.ds(base + j * CHUNK, CHUNK)])
            return ()

        lax.fori_loop(0, n_chunks, chunk_body, ())

    return lookup


def kernel(x, table):
    batch, hist = x.shape
    n_total = batch * hist
    x_flat = x.reshape(NUM_WORKERS, n_total // (NUM_WORKERS * CHUNK), CHUNK)
    x_flat = x_flat.astype(jnp.int32)
    out = _make_lookup(n_total, table.shape[0])(x_flat, table)
    return out.reshape(batch, hist, EMBED_DIM)


# speedup vs baseline: 1.2078x; 1.2078x over previous
import functools

import jax
import jax.numpy as jnp
from jax import lax
from jax.experimental import pallas as pl
from jax.experimental.pallas import tpu as pltpu
from jax.experimental.pallas import tpu_sc as plsc

EMBED_DIM = 64
SCALE = float(EMBED_DIM) ** 0.5

NUM_CORES = 2
NUM_SUBCORES = 16
NUM_WORKERS = NUM_CORES * NUM_SUBCORES
CHUNK = 128
GSIZE = 4
NSETS = 3


def _make_lookup(n_total: int):
    per_worker = n_total // NUM_WORKERS
    n_chunks = per_worker // CHUNK
    n_groups = n_chunks // GSIZE
    assert n_chunks == n_groups * GSIZE
    n_main = (n_groups - 5) // NSETS
    assert 2 + NSETS * n_main == n_groups - 3

    mesh = plsc.VectorSubcoreMesh(
        core_axis_name="c", subcore_axis_name="s",
        num_cores=NUM_CORES, num_subcores=NUM_SUBCORES)

    @functools.partial(
        pl.kernel,
        out_type=jax.ShapeDtypeStruct((n_total, EMBED_DIM), jnp.float32),
        mesh=mesh,
        scratch_types=[
            pltpu.VMEM((n_chunks, CHUNK), jnp.int32),
            [pltpu.VMEM((GSIZE, CHUNK, EMBED_DIM), jnp.float32)
             for _ in range(NSETS)],
            [pltpu.SemaphoreType.DMA for _ in range(NSETS)],
            [pltpu.SemaphoreType.DMA for _ in range(NSETS)],
        ],
        compiler_params=pltpu.CompilerParams(use_tc_tiling_on_sc=False),
    )
    def lookup(x_hbm, table_hbm, out_hbm, idx_v, bufs, gsems, ssems):
        wid = lax.axis_index("s") * NUM_CORES + lax.axis_index("c")
        base = wid * per_worker
        pltpu.sync_copy(x_hbm.at[wid], idx_v)

        def start_gathers(s, g):
            for b in range(GSIZE):
                pltpu.async_copy(
                    table_hbm.at[idx_v.at[g * GSIZE + b]], bufs[s].at[b],
                    gsems[s])

        def wait_gathers(s):
            for b in range(GSIZE):
                pltpu.make_async_copy(
                    table_hbm.at[idx_v.at[b]], bufs[s].at[b],
                    gsems[s]).wait()

        def scale_and_store(s, g):
            buf = bufs[s]
            for b in range(GSIZE):
                @plsc.parallel_loop(0, CHUNK, unroll=8)
                def _(r):
                    for k in range(EMBED_DIM // 16):
                        sl = pl.ds(k * 16, 16)
                        buf[b, r, sl] = buf[b, r, sl] * SCALE
                off = pl.multiple_of(base + (g * GSIZE + b) * CHUNK, CHUNK)
                pltpu.async_copy(buf.at[b], out_hbm.at[pl.ds(off, CHUNK)],
                                 ssems[s])

        def wait_stores(s):
            for b in range(GSIZE):
                pltpu.make_async_copy(
                    bufs[s].at[b],
                    out_hbm.at[pl.ds(pl.multiple_of(base, CHUNK), CHUNK)],
                    ssems[s]).wait()

        start_gathers(0, 0)
        start_gathers(1, 1)
        wait_gathers(0)
        scale_and_store(0, 0)
        start_gathers(2, 2)
        wait_gathers(1)
        scale_and_store(1, 1)

        def main_body(t, _):
            g0 = NSETS * t + 2
            for i, s in enumerate((2, 0, 1)):
                g = g0 + i
                wait_stores((s + 1) % NSETS)
                start_gathers((s + 1) % NSETS, g + 1)
                wait_gathers(s)
                scale_and_store(s, g)
            return ()

        lax.fori_loop(0, n_main, main_body, ())

        gT = n_groups - 3
        wait_stores(0)
        start_gathers(0, gT + 1)
        wait_gathers(2)
        scale_and_store(2, gT)

        wait_stores(1)
        start_gathers(1, gT + 2)
        wait_gathers(0)
        scale_and_store(0, gT + 1)

        wait_stores(2)
        wait_gathers(1)
        scale_and_store(1, gT + 2)

        wait_stores(0)
        wait_stores(1)

    return lookup


def kernel(x, table):
    batch, hist = x.shape
    n_total = batch * hist
    x_flat = x.reshape(NUM_WORKERS, n_total // (NUM_WORKERS * CHUNK), CHUNK)
    x_flat = x_flat.astype(jnp.int32)
    out = _make_lookup(n_total)(x_flat, table)
    return out.reshape(batch, hist, EMBED_DIM)
